# baseline (speedup 1.0000x reference)
import functools

import jax
import jax.numpy as jnp
from jax import lax
from jax.experimental import pallas as pl
from jax.experimental.pallas import tpu as pltpu
from jax.experimental.pallas import tpu_sc as plsc

_N = 1048576
_NC = 2
_NS = 16
_NW = _NC * _NS
_LANES = 16
_ENTRIES = 54

_N_SC = 262144
_N_TC = _N - _N_SC
_BPW = _N_SC // _NW
_P = 2
_CPP = _BPW // _P

_mesh = plsc.VectorSubcoreMesh(core_axis_name="c", subcore_axis_name="s")


@functools.partial(
    pl.kernel,
    out_type=jax.ShapeDtypeStruct((_N_SC,), jnp.float32),
    mesh=_mesh,
    compiler_params=pltpu.CompilerParams(needs_layout_passes=False),
    scratch_types=[
        pltpu.VMEM((_BPW,), jnp.int32),
        pltpu.VMEM((_BPW,), jnp.int32),
        pltpu.VMEM((_BPW,), jnp.float32),
        pltpu.VMEM((_ENTRIES * _LANES,), jnp.float32),
        [pltpu.SemaphoreType.DMA] * (3 * _P + 1),
    ],
)
def _sc_kernel(z_hbm, q_hbm, tab_hbm, out_hbm, z_v, q_v, o_v, tab_v, sems):
    wid = lax.axis_index("s") * _NC + lax.axis_index("c")
    base = wid * _BPW

    tab_cp = pltpu.async_copy(tab_hbm, tab_v, sems[3 * _P])
    in_cps = []
    for p in range(_P):
        off = p * _CPP
        zc = pltpu.async_copy(z_hbm.at[pl.ds(base + off, _CPP)],
                              z_v.at[pl.ds(off, _CPP)], sems[p])
        qc = pltpu.async_copy(q_hbm.at[pl.ds(base + off, _CPP)],
                              q_v.at[pl.ds(off, _CPP)], sems[_P + p])
        in_cps.append((zc, qc))
    tab_cp.wait()

    lane = lax.iota(jnp.int32, _LANES)
    out_cps = []
    for p in range(_P):
        off = p * _CPP
        zc, qc = in_cps[p]
        zc.wait()
        qc.wait()

        @plsc.parallel_loop(off, off + _CPP, step=_LANES, unroll=8)
        def _body(i):
            idx = z_v[pl.ds(i, _LANES)] * 3 + q_v[pl.ds(i, _LANES)]
            slot = idx * _LANES + lane
            o_v[pl.ds(i, _LANES)] = plsc.load_gather(tab_v, [slot])

        out_cps.append(
            pltpu.async_copy(o_v.at[pl.ds(off, _CPP)],
                             out_hbm.at[pl.ds(base + off, _CPP)],
                             sems[2 * _P + p]))
    for cp in out_cps:
        cp.wait()


_ROWS = _N // 128
_ROW_SC = _N_SC // 128
_BR = 512
_GRID_TC = (_ROWS - _ROW_SC) // _BR


def _tc_body(z_ref, q_ref, tab_ref, o_ref):
    idx = z_ref[...] * 3 + q_ref[...]
    x = jnp.broadcast_to(tab_ref[...], idx.shape)
    o_ref[...] = jnp.take_along_axis(x, idx, axis=1, mode="promise_in_bounds")


def kernel(z, charge, energy_table):
    flat = energy_table.reshape(-1)
    tab_rep = jnp.tile(flat.reshape(_ENTRIES, 1), (1, _LANES)).reshape(-1)
    tab128 = jnp.pad(flat, (0, 128 - _ENTRIES)).reshape(1, 128)

    sc_out = _sc_kernel(z, charge, tab_rep)

    z2 = z.reshape(_ROWS, 128)
    q2 = charge.reshape(_ROWS, 128)
    tc_out = pl.pallas_call(
        _tc_body,
        grid=(_GRID_TC,),
        in_specs=[
            pl.BlockSpec((_BR, 128), lambda i: (i + _ROW_SC // _BR, 0)),
            pl.BlockSpec((_BR, 128), lambda i: (i + _ROW_SC // _BR, 0)),
            pl.BlockSpec((1, 128), lambda i: (0, 0)),
        ],
        out_specs=pl.BlockSpec((_BR, 128), lambda i: (i, 0)),
        out_shape=jax.ShapeDtypeStruct((_ROWS - _ROW_SC, 128), jnp.float32),
    )(z2, q2, tab128)

    return jnp.concatenate([sc_out, tc_out.reshape(_N_TC)])


# speedup vs baseline: 1.1866x; 1.1866x over previous
import functools

import jax
import jax.numpy as jnp
from jax import lax
from jax.experimental import pallas as pl
from jax.experimental.pallas import tpu as pltpu
from jax.experimental.pallas import tpu_sc as plsc

_N = 1048576
_NC = 2
_NS = 16
_NW = _NC * _NS
_BPW = _N // _NW
_LANES = 16
_ENTRIES = 54
_P = 4
_CPP = _BPW // _P

_mesh = plsc.VectorSubcoreMesh(core_axis_name="c", subcore_axis_name="s")


@functools.partial(
    pl.kernel,
    out_type=jax.ShapeDtypeStruct((_N,), jnp.float32),
    mesh=_mesh,
    compiler_params=pltpu.CompilerParams(needs_layout_passes=False),
    scratch_types=[
        pltpu.VMEM((_BPW,), jnp.int32),
        pltpu.VMEM((_BPW,), jnp.int32),
        pltpu.VMEM((_BPW,), jnp.float32),
        pltpu.VMEM((18, 3), jnp.float32),
        pltpu.VMEM((_ENTRIES * _LANES,), jnp.float32),
        [pltpu.SemaphoreType.DMA] * (3 * _P + 1),
    ],
)
def _gather_kernel(z_hbm, q_hbm, tab_hbm, out_hbm,
                   z_v, q_v, o_v, tab2_v, rep_v, sems):
    wid = lax.axis_index("s") * _NC + lax.axis_index("c")
    base = wid * _BPW

    tab_cp = pltpu.async_copy(tab_hbm, tab2_v, sems[3 * _P])
    in_cps = []
    for p in range(_P):
        off = p * _CPP
        zc = pltpu.async_copy(z_hbm.at[pl.ds(base + off, _CPP)],
                              z_v.at[pl.ds(off, _CPP)], sems[p])
        qc = pltpu.async_copy(q_hbm.at[pl.ds(base + off, _CPP)],
                              q_v.at[pl.ds(off, _CPP)], sems[_P + p])
        in_cps.append((zc, qc))
    tab_cp.wait()

    lane = lax.iota(jnp.int32, _LANES)
    for k in range((_ENTRIES + _LANES - 1) // _LANES):
        ent = lane + k * _LANES
        m = ent < _ENTRIES
        rows = jnp.where(m, ent // 3, 0)
        cols = jnp.where(m, ent - (ent // 3) * 3, 0)
        vals = plsc.load_gather(tab2_v, [rows, cols])
        for lt in range(_LANES):
            plsc.store_scatter(rep_v, [ent * _LANES + lt], vals, mask=m)
    out_cps = []
    for p in range(_P):
        off = p * _CPP
        zc, qc = in_cps[p]
        zc.wait()
        qc.wait()

        @plsc.parallel_loop(off, off + _CPP, step=_LANES, unroll=8)
        def _body(i):
            idx = z_v[pl.ds(i, _LANES)] * 3 + q_v[pl.ds(i, _LANES)]
            slot = idx * _LANES + lane
            o_v[pl.ds(i, _LANES)] = plsc.load_gather(rep_v, [slot])

        out_cps.append(
            pltpu.async_copy(o_v.at[pl.ds(off, _CPP)],
                             out_hbm.at[pl.ds(base + off, _CPP)],
                             sems[2 * _P + p]))
    for cp in out_cps:
        cp.wait()


def kernel(z, charge, energy_table):
    return _gather_kernel(z, charge, energy_table)


# speedup vs baseline: 1.2339x; 1.0398x over previous
import functools

import jax
import jax.numpy as jnp
from jax import lax
from jax.experimental import pallas as pl
from jax.experimental.pallas import tpu as pltpu
from jax.experimental.pallas import tpu_sc as plsc

_N = 1048576
_NC = 2
_NS = 16
_NW = _NC * _NS
_BPW = _N // _NW
_LANES = 16
_ENTRIES = 54
_P = 8
_CPP = _BPW // _P

_mesh = plsc.VectorSubcoreMesh(core_axis_name="c", subcore_axis_name="s")


@functools.partial(
    pl.kernel,
    out_type=jax.ShapeDtypeStruct((_N,), jnp.float32),
    mesh=_mesh,
    compiler_params=pltpu.CompilerParams(needs_layout_passes=False),
    scratch_types=[
        pltpu.VMEM((_BPW,), jnp.int32),
        pltpu.VMEM((_BPW,), jnp.int32),
        pltpu.VMEM((_BPW,), jnp.float32),
        pltpu.VMEM((_ENTRIES * _LANES,), jnp.float32),
        [pltpu.SemaphoreType.DMA] * (3 * _P + 1),
    ],
)
def _gather_kernel(z_hbm, q_hbm, tab_hbm, out_hbm, z_v, q_v, o_v, tab_v, sems):
    wid = lax.axis_index("s") * _NC + lax.axis_index("c")
    base = wid * _BPW

    tab_cp = pltpu.async_copy(tab_hbm, tab_v, sems[3 * _P])
    in_cps = []
    for p in range(_P):
        off = p * _CPP
        zc = pltpu.async_copy(z_hbm.at[pl.ds(base + off, _CPP)],
                              z_v.at[pl.ds(off, _CPP)], sems[p])
        qc = pltpu.async_copy(q_hbm.at[pl.ds(base + off, _CPP)],
                              q_v.at[pl.ds(off, _CPP)], sems[_P + p])
        in_cps.append((zc, qc))
    tab_cp.wait()

    lane = lax.iota(jnp.int32, _LANES)
    out_cps = []
    for p in range(_P):
        off = p * _CPP
        zc, qc = in_cps[p]
        zc.wait()
        qc.wait()

        @plsc.parallel_loop(off, off + _CPP, step=_LANES, unroll=8)
        def _body(i):
            idx = z_v[pl.ds(i, _LANES)] * 3 + q_v[pl.ds(i, _LANES)]
            slot = idx * _LANES + lane
            o_v[pl.ds(i, _LANES)] = plsc.load_gather(tab_v, [slot])

        out_cps.append(
            pltpu.async_copy(o_v.at[pl.ds(off, _CPP)],
                             out_hbm.at[pl.ds(base + off, _CPP)],
                             sems[2 * _P + p]))
    for cp in out_cps:
        cp.wait()


def kernel(z, charge, energy_table):
    tab_rep = jnp.tile(energy_table.reshape(_ENTRIES, 1), (1, _LANES)).reshape(-1)
    return _gather_kernel(z, charge, tab_rep)
